# baseline (speedup 1.0000x reference)
import functools

import jax
import jax.numpy as jnp
from jax import lax
from jax.experimental import pallas as pl
from jax.experimental.pallas import tpu as pltpu
from jax.experimental.pallas import tpu_sc as plsc

M = 65536
B = 16384
D = 128

NC = 2
NS = 16
NW = NC * NS

HALF = M // NC
PHASES = 8
R = HALF // PHASES
RT = R // NS
TRASH = 8
STRIPE = B // NS
K = 128
KS = 7
NCH_MAX = STRIPE // K
GPW = B // NW
GK = 128


def _mm_body(val_ref, w_ref, out_ref):
    out_ref[...] = jnp.tanh(
        jnp.dot(val_ref[...], w_ref[...],
                preferred_element_type=jnp.float32))


def _project(val, W):
    grid = 16
    rows = B // grid
    return pl.pallas_call(
        _mm_body,
        grid=(grid,),
        in_specs=[
            pl.BlockSpec((rows, D), lambda i: (i, 0)),
            pl.BlockSpec((D, D), lambda i: (0, 0)),
        ],
        out_specs=pl.BlockSpec((rows, D), lambda i: (i, 0)),
        out_shape=jax.ShapeDtypeStruct((B, D), jnp.float32),
    )(val, W)


def _iota16():
    return lax.broadcasted_iota(jnp.int32, (16,), 0)


def _scatter_body(mem_hbm, proj_hbm, idx_hbm, out_hbm,
                  idx_v, plist3d, lidx3d, projbufs, chunks, ldsems, wbsems,
                  psems, sem):
    c = lax.axis_index("c")
    s = lax.axis_index("s")
    stripe_base = s * STRIPE

    pltpu.sync_copy(idx_hbm.at[pl.ds(stripe_base, STRIPE)], idx_v)

    def chunk_base(p):
        return c * HALF + p * R

    def start_load(p):
        return pltpu.async_copy(
            mem_hbm.at[pl.ds(chunk_base(p) + s * RT, RT)],
            chunks[p % 2].at[pl.ds(s * RT, RT)],
            ldsems[p % 2])

    loads = [start_load(0), start_load(1)]
    wbs = [None, None]

    nchs = []
    for p in range(PHASES):
        lo = chunk_base(p)
        pl2 = plist3d.at[p]
        li2 = lidx3d.at[p]

        def compress(v, cnt):
            idxv = idx_v[pl.ds(v * 16, 16)]
            lid = idxv - lo
            m = (lid >= 0) & (lid < R)
            posv = stripe_base + v * 16 + _iota16()
            mi = jnp.where(m, 1, 0)
            tgt = cnt + plsc.cumsum(mi) - 1
            plsc.store_scatter(pl2, [tgt >> KS, tgt & (K - 1)], posv,
                               mask=m)
            plsc.store_scatter(li2, [tgt >> KS, tgt & (K - 1)], lid,
                               mask=m)
            return cnt + jnp.sum(mi, axis=0)
        cnt = lax.fori_loop(0, STRIPE // 16, compress, jnp.int32(0))
        nch = (cnt + (K - 1)) // K

        def padtail(q, _):
            t = cnt + q * 16 + _iota16()
            tm = t < nch * K
            plsc.store_scatter(pl2, [t >> KS, t & (K - 1)],
                               stripe_base + _iota16() * 64, mask=tm)
            plsc.store_scatter(li2, [t >> KS, t & (K - 1)],
                               R + (_iota16() & (TRASH - 1)), mask=tm)
            return 0
        lax.fori_loop(0, (nch * K - cnt + 15) // 16, padtail, 0)
        nchs.append(nch)

    def prefetch(p):
        @pl.when(nchs[p] > 0)
        def _():
            pltpu.async_copy(proj_hbm.at[plist3d.at[p].at[0]],
                             projbufs[p % 2], psems[p % 2])

    prefetch(0)

    for p in range(PHASES):
        buf = p % 2
        chunk = chunks[buf]
        lo = chunk_base(p)
        nch = nchs[p]
        pbuf = projbufs[p % 2]

        if p + 1 < PHASES:
            prefetch(p + 1)

        loads[buf].wait()
        plsc.subcore_barrier()

        @pl.when(nch > 0)
        def _():
            pltpu.make_async_copy(proj_hbm.at[plist3d.at[p].at[0]],
                                  pbuf, psems[p % 2]).wait()
            pltpu.sync_copy(pbuf, chunk.at[lidx3d.at[p].at[0]], add=True)

        def scat_body(j, _):
            pltpu.async_copy(
                proj_hbm.at[plist3d.at[p].at[j]], pbuf, sem).wait()
            pltpu.sync_copy(pbuf, chunk.at[lidx3d.at[p].at[j]], add=True)
            return 0
        lax.fori_loop(1, nch, scat_body, 0)

        plsc.subcore_barrier()

        if 1 <= p < PHASES - 1:
            nbuf = (p + 1) % 2
            wbs[nbuf].wait()
            loads[nbuf] = start_load(p + 1)
        wbs[buf] = pltpu.async_copy(
            chunk.at[pl.ds(s * RT, RT)],
            out_hbm.at[pl.ds(lo + s * RT, RT)],
            wbsems[buf])

    wbs[0].wait()
    wbs[1].wait()


def _scatter(mem, proj, idx):
    mesh = plsc.VectorSubcoreMesh(core_axis_name="c", subcore_axis_name="s")
    return pl.kernel(
        _scatter_body,
        mesh=mesh,
        out_type=jax.ShapeDtypeStruct((M, D), jnp.float32),
        scratch_types=[
            pltpu.VMEM((STRIPE,), jnp.int32),
            pltpu.VMEM((PHASES, NCH_MAX, K), jnp.int32),
            pltpu.VMEM((PHASES, NCH_MAX, K), jnp.int32),
            [pltpu.VMEM((K, D), jnp.float32)
             for _ in range(2)],
            [pltpu.VMEM_SHARED((R + TRASH, D), jnp.float32)
             for _ in range(2)],
            [pltpu.SemaphoreType.DMA for _ in range(2)],
            [pltpu.SemaphoreType.DMA for _ in range(2)],
            [pltpu.SemaphoreType.DMA for _ in range(2)],
            pltpu.SemaphoreType.DMA,
        ],
        compiler_params=pltpu.CompilerParams(needs_layout_passes=False),
    )(mem, proj, idx)


def _gather_body(newmem_hbm, idx_hbm, read_hbm, idx_v, rowbufs, sems):
    c = lax.axis_index("c")
    s = lax.axis_index("s")
    wid = s * NC + c
    base = wid * GPW
    pltpu.sync_copy(idx_hbm.at[pl.ds(base, GPW)], idx_v)
    n = GPW // GK
    copies = [None, None]
    for j in range(n):
        copies[j % 2] = pltpu.async_copy(
            newmem_hbm.at[idx_v.at[pl.ds(j * GK, GK)]], rowbufs[j % 2],
            sems[j % 2])
        if j >= 1:
            copies[(j - 1) % 2].wait()
            pltpu.sync_copy(rowbufs[(j - 1) % 2],
                            read_hbm.at[pl.ds(base + (j - 1) * GK, GK)])
    copies[(n - 1) % 2].wait()
    pltpu.sync_copy(rowbufs[(n - 1) % 2],
                    read_hbm.at[pl.ds(base + (n - 1) * GK, GK)])


def _gather(new_mem, idx):
    mesh = plsc.VectorSubcoreMesh(core_axis_name="c", subcore_axis_name="s")
    return pl.kernel(
        _gather_body,
        mesh=mesh,
        out_type=jax.ShapeDtypeStruct((B, D), jnp.float32),
        scratch_types=[
            pltpu.VMEM((GPW,), jnp.int32),
            [pltpu.VMEM((GK, D), jnp.float32) for _ in range(2)],
            [pltpu.SemaphoreType.DMA for _ in range(2)],
        ],
        compiler_params=pltpu.CompilerParams(needs_layout_passes=False),
    )(new_mem, idx)


def kernel(mem, val, idx, W):
    proj = _project(val, W)
    new_mem = _scatter(mem, proj, idx.astype(jnp.int32))
    read = _gather(new_mem, idx.astype(jnp.int32))
    return new_mem, read


# speedup vs baseline: 1.0567x; 1.0567x over previous
import functools

import jax
import jax.numpy as jnp
from jax import lax
from jax.experimental import pallas as pl
from jax.experimental.pallas import tpu as pltpu
from jax.experimental.pallas import tpu_sc as plsc

M = 65536
B = 16384
D = 128

NC = 2
NS = 16
NW = NC * NS

HALF = M // NC
PHASES = 8
R = HALF // PHASES
RT = R // NS
TRASH = 8
STRIPE = B // NS
K = 128
KS = 7
NCH_MAX = STRIPE // K
GPW = B // NW
GK = 128


def _mm_body(val_ref, w_ref, out_ref):
    out_ref[...] = jnp.tanh(
        jnp.dot(val_ref[...], w_ref[...],
                preferred_element_type=jnp.float32))


def _project(val, W):
    grid = 16
    rows = B // grid
    return pl.pallas_call(
        _mm_body,
        grid=(grid,),
        in_specs=[
            pl.BlockSpec((rows, D), lambda i: (i, 0)),
            pl.BlockSpec((D, D), lambda i: (0, 0)),
        ],
        out_specs=pl.BlockSpec((rows, D), lambda i: (i, 0)),
        out_shape=jax.ShapeDtypeStruct((B, D), jnp.float32),
    )(val, W)


def _iota16():
    return lax.broadcasted_iota(jnp.int32, (16,), 0)


def _scatter_body(mem_hbm, proj_hbm, idx_hbm, out_hbm,
                  idx_v, plist3d, lidx3d, projbufs, chunks, ldsems, wbsems,
                  psems, sem):
    c = lax.axis_index("c")
    s = lax.axis_index("s")
    stripe_base = s * STRIPE

    pltpu.sync_copy(idx_hbm.at[pl.ds(stripe_base, STRIPE)], idx_v)

    def chunk_base(p):
        return c * HALF + p * R

    def start_load(p):
        return pltpu.async_copy(
            mem_hbm.at[pl.ds(chunk_base(p) + s * RT, RT)],
            chunks[p % 2].at[pl.ds(s * RT, RT)],
            ldsems[p % 2])

    loads = [start_load(0), start_load(1)]
    wbs = [None, None]

    nchs = [None] * PHASES

    def do_compress(p):
        lo = chunk_base(p)
        pl2 = plist3d.at[p]
        li2 = lidx3d.at[p]

        def compress(v, cnt):
            idxv = idx_v[pl.ds(v * 16, 16)]
            lid = idxv - lo
            m = (lid >= 0) & (lid < R)
            posv = stripe_base + v * 16 + _iota16()
            mi = jnp.where(m, 1, 0)
            tgt = cnt + plsc.cumsum(mi) - 1
            plsc.store_scatter(pl2, [tgt >> KS, tgt & (K - 1)], posv,
                               mask=m)
            plsc.store_scatter(li2, [tgt >> KS, tgt & (K - 1)], lid,
                               mask=m)
            return cnt + jnp.sum(mi, axis=0)
        cnt = lax.fori_loop(0, STRIPE // 16, compress, jnp.int32(0))
        nch = (cnt + (K - 1)) // K

        def padtail(q, _):
            t = cnt + q * 16 + _iota16()
            tm = t < nch * K
            plsc.store_scatter(pl2, [t >> KS, t & (K - 1)],
                               stripe_base + _iota16() * 64, mask=tm)
            plsc.store_scatter(li2, [t >> KS, t & (K - 1)],
                               R + (_iota16() & (TRASH - 1)), mask=tm)
            return 0
        lax.fori_loop(0, (nch * K - cnt + 15) // 16, padtail, 0)
        nchs[p] = nch

    do_compress(0)
    do_compress(1)

    def prefetch(p):
        @pl.when(nchs[p] > 0)
        def _():
            pltpu.async_copy(proj_hbm.at[plist3d.at[p].at[0]],
                             projbufs[p % 2], psems[p % 2])

    prefetch(0)

    for p in range(PHASES):
        buf = p % 2
        chunk = chunks[buf]
        lo = chunk_base(p)
        nch = nchs[p]
        pbuf = projbufs[p % 2]

        if p + 1 < PHASES:
            prefetch(p + 1)
        if p + 2 < PHASES:
            do_compress(p + 2)

        loads[buf].wait()
        plsc.subcore_barrier()

        @pl.when(nch > 0)
        def _():
            pltpu.make_async_copy(proj_hbm.at[plist3d.at[p].at[0]],
                                  pbuf, psems[p % 2]).wait()
            pltpu.sync_copy(pbuf, chunk.at[lidx3d.at[p].at[0]], add=True)

        def scat_body(j, _):
            pltpu.async_copy(
                proj_hbm.at[plist3d.at[p].at[j]], pbuf, sem).wait()
            pltpu.sync_copy(pbuf, chunk.at[lidx3d.at[p].at[j]], add=True)
            return 0
        lax.fori_loop(1, nch, scat_body, 0)

        plsc.subcore_barrier()

        if 1 <= p < PHASES - 1:
            nbuf = (p + 1) % 2
            wbs[nbuf].wait()
            loads[nbuf] = start_load(p + 1)
        wbs[buf] = pltpu.async_copy(
            chunk.at[pl.ds(s * RT, RT)],
            out_hbm.at[pl.ds(lo + s * RT, RT)],
            wbsems[buf])

    wbs[0].wait()
    wbs[1].wait()


def _scatter(mem, proj, idx):
    mesh = plsc.VectorSubcoreMesh(core_axis_name="c", subcore_axis_name="s")
    return pl.kernel(
        _scatter_body,
        mesh=mesh,
        out_type=jax.ShapeDtypeStruct((M, D), jnp.float32),
        scratch_types=[
            pltpu.VMEM((STRIPE,), jnp.int32),
            pltpu.VMEM((PHASES, NCH_MAX, K), jnp.int32),
            pltpu.VMEM((PHASES, NCH_MAX, K), jnp.int32),
            [pltpu.VMEM((K, D), jnp.float32)
             for _ in range(2)],
            [pltpu.VMEM_SHARED((R + TRASH, D), jnp.float32)
             for _ in range(2)],
            [pltpu.SemaphoreType.DMA for _ in range(2)],
            [pltpu.SemaphoreType.DMA for _ in range(2)],
            [pltpu.SemaphoreType.DMA for _ in range(2)],
            pltpu.SemaphoreType.DMA,
        ],
        compiler_params=pltpu.CompilerParams(needs_layout_passes=False),
    )(mem, proj, idx)


def _gather_body(newmem_hbm, idx_hbm, read_hbm, idx_v, rowbufs, sems):
    c = lax.axis_index("c")
    s = lax.axis_index("s")
    wid = s * NC + c
    base = wid * GPW
    pltpu.sync_copy(idx_hbm.at[pl.ds(base, GPW)], idx_v)
    n = GPW // GK
    copies = [None, None]
    for j in range(n):
        copies[j % 2] = pltpu.async_copy(
            newmem_hbm.at[idx_v.at[pl.ds(j * GK, GK)]], rowbufs[j % 2],
            sems[j % 2])
        if j >= 1:
            copies[(j - 1) % 2].wait()
            pltpu.sync_copy(rowbufs[(j - 1) % 2],
                            read_hbm.at[pl.ds(base + (j - 1) * GK, GK)])
    copies[(n - 1) % 2].wait()
    pltpu.sync_copy(rowbufs[(n - 1) % 2],
                    read_hbm.at[pl.ds(base + (n - 1) * GK, GK)])


def _gather(new_mem, idx):
    mesh = plsc.VectorSubcoreMesh(core_axis_name="c", subcore_axis_name="s")
    return pl.kernel(
        _gather_body,
        mesh=mesh,
        out_type=jax.ShapeDtypeStruct((B, D), jnp.float32),
        scratch_types=[
            pltpu.VMEM((GPW,), jnp.int32),
            [pltpu.VMEM((GK, D), jnp.float32) for _ in range(2)],
            [pltpu.SemaphoreType.DMA for _ in range(2)],
        ],
        compiler_params=pltpu.CompilerParams(needs_layout_passes=False),
    )(new_mem, idx)


def kernel(mem, val, idx, W):
    proj = _project(val, W)
    new_mem = _scatter(mem, proj, idx.astype(jnp.int32))
    read = _gather(new_mem, idx.astype(jnp.int32))
    return new_mem, read
